# baseline (speedup 1.0000x reference)
import functools

import jax
import jax.numpy as jnp
from jax import lax
from jax.experimental import pallas as pl
from jax.experimental.pallas import tpu as pltpu
from jax.experimental.pallas import tpu_sc as plsc

_ROWS = 512
_NW = 32
_CHUNK = 16384
_LANES = 16


def _add_kernel(x_ref, pos_ref, out_ref):
    out_ref[...] = x_ref[...] + pos_ref[...][None, :, :]


def _sc_body(x_hbm, pos_hbm, out_hbm, xbuf, pbuf, obuf, *, per_w, x_base):
    wid = lax.axis_index("s") * 2 + lax.axis_index("c")
    xoff = pl.multiple_of(x_base + wid * per_w, _CHUNK)
    ooff = pl.multiple_of(wid * per_w, _CHUNK)
    n_chunks = per_w // _CHUNK

    def chunk_body(k, carry):
        base = pl.multiple_of(xoff + k * _CHUNK, _CHUNK)
        obase = pl.multiple_of(ooff + k * _CHUNK, _CHUNK)
        pltpu.sync_copy(x_hbm.at[pl.ds(base, _CHUNK)], xbuf)
        pltpu.sync_copy(pos_hbm.at[pl.ds(obase, _CHUNK)], pbuf)

        def add_body(j, c):
            sl = pl.ds(j * _LANES, _LANES)
            obuf[sl] = xbuf[sl] + pbuf[sl]
            return c

        lax.fori_loop(0, _CHUNK // _LANES, add_body, 0)
        pltpu.sync_copy(obuf, out_hbm.at[pl.ds(obase, _CHUNK)])
        return carry

    lax.fori_loop(0, n_chunks, chunk_body, 0)


def kernel(x, position_embeddings):
    B, T, C = x.shape
    pos = position_embeddings[:T]

    tc_out = pl.pallas_call(
        _add_kernel,
        grid=(T // _ROWS,),
        in_specs=[
            pl.BlockSpec((B - 1, _ROWS, C), lambda t: (0, t, 0)),
            pl.BlockSpec((_ROWS, C), lambda t: (t, 0)),
        ],
        out_specs=pl.BlockSpec((B - 1, _ROWS, C), lambda t: (0, t, 0)),
        out_shape=jax.ShapeDtypeStruct((B - 1, T, C), x.dtype),
    )(x, pos)

    pos_words = T * C
    per_w = pos_words // _NW
    mesh = plsc.VectorSubcoreMesh(core_axis_name="c", subcore_axis_name="s")
    sc_call = pl.kernel(
        functools.partial(_sc_body, per_w=per_w, x_base=(B - 1) * T * C),
        mesh=mesh,
        out_type=jax.ShapeDtypeStruct((pos_words,), jnp.float32),
        scratch_types=[
            pltpu.VMEM((_CHUNK,), jnp.float32),
            pltpu.VMEM((_CHUNK,), jnp.float32),
            pltpu.VMEM((_CHUNK,), jnp.float32),
        ],
    )
    sc_out = sc_call(x.reshape(-1), pos.reshape(-1)).reshape(1, T, C)

    return jnp.concatenate([tc_out, sc_out], axis=0)


# speedup vs baseline: 4.3603x; 4.3603x over previous
import jax
import jax.numpy as jnp
from jax.experimental import pallas as pl


_ROWS = 256


def _add_kernel(x_ref, pos_ref, out_ref):
    t = pl.program_id(0)
    out_ref[...] = x_ref[...] + pos_ref[pl.ds(t * _ROWS, _ROWS), :][None, :, :]


def kernel(x, position_embeddings):
    B, T, C = x.shape
    pos = position_embeddings[:T]
    grid = (T // _ROWS,)
    return pl.pallas_call(
        _add_kernel,
        grid=grid,
        in_specs=[
            pl.BlockSpec((B, _ROWS, C), lambda t: (0, t, 0)),
            pl.BlockSpec((T, C), lambda t: (0, 0)),
        ],
        out_specs=pl.BlockSpec((B, _ROWS, C), lambda t: (0, t, 0)),
        out_shape=jax.ShapeDtypeStruct((B, T, C), x.dtype),
    )(x, pos)
